# Initial kernel scaffold; baseline (speedup 1.0000x reference)
import jax
import jax.numpy as jnp
from jax import lax
from jax.experimental import pallas as pl
from jax.experimental.pallas import tpu as pltpu
from jax.experimental.pallas import tpu_sc as plsc

N = 10000
E = 320000
D = 128

NC, NS = 2, 16
NW = NC * NS
EPW = E // NW
CH = 400
NCH = EPW // CH
BI = 80
KI = CH // BI
RPT = N // NS


BE = 2560


def _pe_body(ew_ref, w1_ref, w2_ref, out_ref):
    ew = ew_ref[...]
    w1 = w1_ref[...]
    h = ew[:, 0:1] * w1[0:1, :] + ew[:, 1:2] * w1[1:2, :]
    h = jnp.maximum(h, 0.0)
    out_ref[...] = (
        jnp.dot(h, w2_ref[...], preferred_element_type=jnp.float32) + 1.0
    )


def _pe_call(edge_w, W1, W2):
    return pl.pallas_call(
        _pe_body,
        grid=(E // BE,),
        in_specs=[
            pl.BlockSpec((BE, 2), lambda i: (i, 0)),
            pl.BlockSpec((2, D // 2), lambda i: (0, 0)),
            pl.BlockSpec((D // 2, D), lambda i: (0, 0)),
        ],
        out_specs=pl.BlockSpec((BE, D), lambda i: (i, 0)),
        out_shape=jax.ShapeDtypeStruct((E, D), jnp.float32),
    )(edge_w, W1, W2)


def _sc_body(x_hbm, src_hbm, dst_hbm, pe_hbm, acc_hbm, cnt_hbm,
             src_v, dst_v, pe_v, xr_v, cnt_v, acc_sh, sem):
    c = lax.axis_index("c")
    s = lax.axis_index("s")
    w = s * NC + c

    zeros16 = jnp.zeros((16,), jnp.float32)
    ones16 = jnp.ones((16,), jnp.float32)

    @pl.loop(0, N // 16)
    def _(i):
        cnt_v[pl.ds(i * 16, 16)] = zeros16

    @pl.loop(0, CH)
    def _(r):
        for g in range(D // 16):
            xr_v[r, pl.ds(g * 16, 16)] = zeros16

    pltpu.sync_copy(xr_v, acc_sh.at[pl.ds(s * RPT, CH)])
    pltpu.sync_copy(xr_v.at[pl.ds(0, RPT - CH)],
                    acc_sh.at[pl.ds(s * RPT + CH, RPT - CH)])
    plsc.subcore_barrier()

    @pl.loop(0, NCH)
    def _(j):
        ebase = w * EPW + j * CH
        rbase = w * (EPW // BI) + j * KI
        pltpu.sync_copy(src_hbm.at[pl.ds(rbase, KI)], src_v)
        pltpu.sync_copy(dst_hbm.at[pl.ds(rbase, KI)], dst_v)
        cps = [
            pltpu.async_copy(x_hbm.at[src_v.at[t]],
                             xr_v.at[pl.ds(t * BI, BI)], sem)
            for t in range(KI)
        ]
        pltpu.sync_copy(pe_hbm.at[pl.ds(ebase, CH)], pe_v)
        for cp in cps:
            cp.wait()

        @pl.loop(0, CH)
        def _(r):
            for g in range(D // 16):
                sl = pl.ds(g * 16, 16)
                xr_v[r, sl] = xr_v[r, sl] * pe_v[r, sl]

        for t in range(KI):
            pltpu.sync_copy(xr_v.at[pl.ds(t * BI, BI)],
                            acc_sh.at[dst_v.at[t]], add=True)

        for t in range(KI):
            for g in range(BI // 16):
                idx16 = dst_v[t, pl.ds(g * 16, 16)]
                plsc.addupdate_scatter(cnt_v, [idx16], ones16)

    plsc.subcore_barrier()

    pltpu.sync_copy(acc_sh.at[pl.ds(s * RPT, RPT)],
                    acc_hbm.at[c, pl.ds(s * RPT, RPT)])
    pltpu.sync_copy(cnt_v, cnt_hbm.at[w])


_sc_call = pl.kernel(
    _sc_body,
    out_type=[
        jax.ShapeDtypeStruct((NC, N, D), jnp.float32),
        jax.ShapeDtypeStruct((NW, N), jnp.float32),
    ],
    mesh=plsc.VectorSubcoreMesh(core_axis_name="c", subcore_axis_name="s"),
    scratch_types=[
        pltpu.VMEM((KI, BI), jnp.int32),
        pltpu.VMEM((KI, BI), jnp.int32),
        pltpu.VMEM((CH, D), jnp.float32),
        pltpu.VMEM((CH, D), jnp.float32),
        pltpu.VMEM((N,), jnp.float32),
        pltpu.VMEM_SHARED((N, D), jnp.float32),
        pltpu.SemaphoreType.DMA,
    ],
)


BN = 2000
NB = N // BN


def _fin_body(acc_ref, cnt_ref, x_ref, w_ref, b_ref, g_ref, be_ref,
              out_ref, t_sc, s1_sc, s2_sc):
    p = pl.program_id(0)
    i = pl.program_id(1)

    @pl.when(p == 0)
    def _():
        xb = x_ref[...]
        ssum = acc_ref[0] + acc_ref[1] + xb
        cnt = jnp.sum(cnt_ref[...], axis=0) + 1.0
        mean = ssum / cnt[:, None]
        wt = w_ref[...]
        t_pre = (
            jnp.dot(xb, wt[:D], preferred_element_type=jnp.float32)
            + jnp.dot(mean, wt[D:], preferred_element_type=jnp.float32)
            + b_ref[...][None, :]
        )
        nrm = jnp.sqrt(jnp.sum(t_pre * t_pre, axis=1, keepdims=True))
        t = t_pre / jnp.maximum(nrm, 1e-12)
        t_sc[pl.ds(i * BN, BN), :] = t

        @pl.when(i == 0)
        def _():
            s1_sc[...] = jnp.zeros_like(s1_sc)
            s2_sc[...] = jnp.zeros_like(s2_sc)

        s1_sc[...] += jnp.sum(t, axis=0, keepdims=True)
        s2_sc[...] += jnp.sum(t * t, axis=0, keepdims=True)

    @pl.when(p == 1)
    def _():
        t = t_sc[pl.ds(i * BN, BN), :]
        mu = s1_sc[...] / N
        var = s2_sc[...] / N - mu * mu
        y = (t - mu) * lax.rsqrt(var + 1e-5) * g_ref[...][None, :] \
            + be_ref[...][None, :]
        out_ref[...] = jnp.maximum(y + x_ref[...], 0.0)


def _fin_call(acc, cntp, x, W, b, gamma, beta):
    return pl.pallas_call(
        _fin_body,
        grid=(2, NB),
        in_specs=[
            pl.BlockSpec((NC, BN, D), lambda p, i: (0, i, 0)),
            pl.BlockSpec((NW, BN), lambda p, i: (0, i)),
            pl.BlockSpec((BN, D), lambda p, i: (i, 0)),
            pl.BlockSpec((2 * D, D), lambda p, i: (0, 0)),
            pl.BlockSpec((D,), lambda p, i: (0,)),
            pl.BlockSpec((D,), lambda p, i: (0,)),
            pl.BlockSpec((D,), lambda p, i: (0,)),
        ],
        out_specs=pl.BlockSpec((BN, D), lambda p, i: (i, 0)),
        out_shape=jax.ShapeDtypeStruct((N, D), jnp.float32),
        scratch_shapes=[
            pltpu.VMEM((N, D), jnp.float32),
            pltpu.VMEM((1, D), jnp.float32),
            pltpu.VMEM((1, D), jnp.float32),
        ],
    )(acc, cntp, x, W, b, gamma, beta)


def kernel(x, edge_index, edge_w, W1, W2, W, b, gamma, beta):
    src2d = edge_index[0].reshape(E // BI, BI)
    dst2d = edge_index[1].reshape(E // BI, BI)
    pe1 = _pe_call(edge_w, W1, W2)
    acc, cntp = _sc_call(x, src2d, dst2d, pe1)
    return _fin_call(acc, cntp, x, W, b, gamma, beta)


# speedup vs baseline: 2.1698x; 2.1698x over previous
import jax
import jax.numpy as jnp
from jax import lax
from jax.experimental import pallas as pl
from jax.experimental.pallas import tpu as pltpu
from jax.experimental.pallas import tpu_sc as plsc

N = 10000
E = 320000
D = 128
DH = D // 2

NC, NS = 2, 16
EP = 327680
NP = 10240
BI = 128
RW = EP // BI // NS
G = 8
NG = RW // G
SCH = 256
NU = G * BI // SCH
RPS = SCH // BI
RPT = NP // NS


BE = 2560


def _pe_body(ew_ref, w1_ref, w2_ref, out_ref):
    ew = ew_ref[...]
    w1 = w1_ref[...]
    h = ew[:, 0:1] * w1[0:1, :] + ew[:, 1:2] * w1[1:2, :]
    h = jnp.maximum(h, 0.0)
    out_ref[0] = (
        jnp.dot(h, w2_ref[0], preferred_element_type=jnp.float32) + 1.0
    )


def _pe_call(edge_w, W1, W2):
    return pl.pallas_call(
        _pe_body,
        grid=(EP // BE, NC),
        in_specs=[
            pl.BlockSpec((BE, 2), lambda i, h: (i, 0)),
            pl.BlockSpec((2, DH), lambda i, h: (0, 0)),
            pl.BlockSpec((1, DH, DH), lambda i, h: (h, 0, 0)),
        ],
        out_specs=pl.BlockSpec((1, BE, DH), lambda i, h: (h, i, 0)),
        out_shape=jax.ShapeDtypeStruct((NC, EP, DH), jnp.float32),
    )(edge_w, W1, W2)


def _sc_body(x_hbm, src_hbm, dst_hbm, pe_hbm, acc_hbm, cnt_hbm,
             src_v, dst_v, gidx_v, pe_v, xr_v, cnt_v, acc_sh, sem):
    c = lax.axis_index("c")
    s = lax.axis_index("s")

    zeros16 = jnp.zeros((16,), jnp.float32)
    ones16 = jnp.ones((16,), jnp.float32)

    @pl.loop(0, NP // 16)
    def _(i):
        cnt_v[pl.ds(i * 16, 16)] = zeros16

    @pl.loop(0, SCH)
    def _(r):
        for g in range(DH // 16):
            xr_v[r, pl.ds(g * 16, 16)] = zeros16

    pltpu.sync_copy(xr_v, acc_sh.at[pl.ds(s * RPT, SCH)])
    pltpu.sync_copy(xr_v, acc_sh.at[pl.ds(s * RPT + SCH, SCH)])
    pltpu.sync_copy(xr_v.at[pl.ds(0, RPT - 2 * SCH)],
                    acc_sh.at[pl.ds(s * RPT + 2 * SCH, RPT - 2 * SCH)])
    plsc.subcore_barrier()

    @pl.loop(0, NG)
    def _(j):
        rbase = s * RW + j * G
        pltpu.sync_copy(src_hbm.at[pl.ds(rbase, G)], src_v)
        pltpu.sync_copy(dst_hbm.at[pl.ds(rbase, G)], dst_v)

        @pl.loop(0, G)
        def _(r):
            for g in range(BI // 16):
                sl = pl.ds(g * 16, 16)
                gidx_v[r, sl] = src_v[r, sl] + c * N

        for u in range(NU):
            ebase = (rbase + u * RPS) * BI
            cps = [
                pltpu.async_copy(x_hbm.at[gidx_v.at[u * RPS + t]],
                                 xr_v.at[pl.ds(t * BI, BI)], sem)
                for t in range(RPS)
            ]
            pltpu.sync_copy(pe_hbm.at[c, pl.ds(ebase, SCH)], pe_v)
            for cp in cps:
                cp.wait()

            @pl.loop(0, SCH)
            def _(r):
                for g in range(DH // 16):
                    sl = pl.ds(g * 16, 16)
                    xr_v[r, sl] = xr_v[r, sl] * pe_v[r, sl]

            for t in range(RPS):
                pltpu.sync_copy(xr_v.at[pl.ds(t * BI, BI)],
                                acc_sh.at[dst_v.at[u * RPS + t]], add=True)

            @pl.when(c == 0)
            def _():
                for t in range(RPS):
                    for g in range(BI // 16):
                        idx16 = dst_v[u * RPS + t, pl.ds(g * 16, 16)]
                        plsc.addupdate_scatter(cnt_v, [idx16], ones16)

    plsc.subcore_barrier()

    pltpu.sync_copy(acc_sh.at[pl.ds(s * RPT, RPT)],
                    acc_hbm.at[c, pl.ds(s * RPT, RPT)])

    @pl.when(c == 0)
    def _():
        pltpu.sync_copy(cnt_v, cnt_hbm.at[pl.ds(s * NP, NP)])


_sc_call = pl.kernel(
    _sc_body,
    out_type=[
        jax.ShapeDtypeStruct((NC, NP, DH), jnp.float32),
        jax.ShapeDtypeStruct((NS * NP,), jnp.float32),
    ],
    mesh=plsc.VectorSubcoreMesh(core_axis_name="c", subcore_axis_name="s"),
    compiler_params=pltpu.CompilerParams(needs_layout_passes=False,
                                         use_tc_tiling_on_sc=False),
    scratch_types=[
        pltpu.VMEM((G, BI), jnp.int32),
        pltpu.VMEM((G, BI), jnp.int32),
        pltpu.VMEM((G, BI), jnp.int32),
        pltpu.VMEM((SCH, DH), jnp.float32),
        pltpu.VMEM((SCH, DH), jnp.float32),
        pltpu.VMEM((NP,), jnp.float32),
        pltpu.VMEM_SHARED((NP, DH), jnp.float32),
        pltpu.SemaphoreType.DMA,
    ],
)


BN = 2000
NB = N // BN


def _fin_body(acc_ref, cnt_ref, x_ref, w_ref, b_ref, g_ref, be_ref,
              out_ref, t_sc, s1_sc, s2_sc):
    p = pl.program_id(0)
    i = pl.program_id(1)

    @pl.when(p == 0)
    def _():
        xb = x_ref[...]
        ssum = jnp.concatenate([acc_ref[0], acc_ref[1]], axis=1) + xb
        cnt = jnp.sum(cnt_ref[...], axis=1) + 1.0
        mean = ssum / cnt[:, None]
        wt = w_ref[...]
        t_pre = (
            jnp.dot(xb, wt[:D], preferred_element_type=jnp.float32)
            + jnp.dot(mean, wt[D:], preferred_element_type=jnp.float32)
            + b_ref[...][None, :]
        )
        nrm = jnp.sqrt(jnp.sum(t_pre * t_pre, axis=1, keepdims=True))
        t = t_pre / jnp.maximum(nrm, 1e-12)
        t_sc[pl.ds(i * BN, BN), :] = t

        @pl.when(i == 0)
        def _():
            s1_sc[...] = jnp.zeros_like(s1_sc)
            s2_sc[...] = jnp.zeros_like(s2_sc)

        s1_sc[...] += jnp.sum(t, axis=0, keepdims=True)
        s2_sc[...] += jnp.sum(t * t, axis=0, keepdims=True)

    @pl.when(p == 1)
    def _():
        t = t_sc[pl.ds(i * BN, BN), :]
        mu = s1_sc[...] / N
        var = s2_sc[...] / N - mu * mu
        y = (t - mu) * lax.rsqrt(var + 1e-5) * g_ref[...][None, :] \
            + be_ref[...][None, :]
        out_ref[...] = jnp.maximum(y + x_ref[...], 0.0)


def _fin_call(acc, cntp, x, W, b, gamma, beta):
    return pl.pallas_call(
        _fin_body,
        grid=(2, NB),
        in_specs=[
            pl.BlockSpec((NC, BN, DH), lambda p, i: (0, i, 0)),
            pl.BlockSpec((BN, NS), lambda p, i: (i, 0)),
            pl.BlockSpec((BN, D), lambda p, i: (i, 0)),
            pl.BlockSpec((2 * D, D), lambda p, i: (0, 0)),
            pl.BlockSpec((D,), lambda p, i: (0,)),
            pl.BlockSpec((D,), lambda p, i: (0,)),
            pl.BlockSpec((D,), lambda p, i: (0,)),
        ],
        out_specs=pl.BlockSpec((BN, D), lambda p, i: (i, 0)),
        out_shape=jax.ShapeDtypeStruct((N, D), jnp.float32),
        scratch_shapes=[
            pltpu.VMEM((N, D), jnp.float32),
            pltpu.VMEM((1, D), jnp.float32),
            pltpu.VMEM((1, D), jnp.float32),
        ],
    )(acc, cntp, x, W, b, gamma, beta)


def kernel(x, edge_index, edge_w, W1, W2, W, b, gamma, beta):
    pad = EP - E
    src2d = jnp.concatenate(
        [edge_index[0], jnp.zeros((pad,), jnp.int32)]).reshape(EP // BI, BI)
    dst2d = jnp.concatenate(
        [edge_index[1], jnp.full((pad,), N, jnp.int32)]).reshape(EP // BI, BI)
    ew_pad = jnp.concatenate([edge_w, jnp.zeros((pad, 2), jnp.float32)])
    xcat = jnp.concatenate([x[:, :DH], x[:, DH:]], axis=0)
    w2s = jnp.stack([W2[:, :DH], W2[:, DH:]])
    pe1 = _pe_call(ew_pad, W1, w2s)
    acc, cntp = _sc_call(xcat, src2d, dst2d, pe1)
    return _fin_call(acc, cntp.reshape(NS, NP).T, x, W, b, gamma, beta)


# speedup vs baseline: 2.4403x; 1.1247x over previous
import jax
import jax.numpy as jnp
from jax import lax
from jax.experimental import pallas as pl
from jax.experimental.pallas import tpu as pltpu
from jax.experimental.pallas import tpu_sc as plsc

N = 10000
E = 320000
D = 128
DH = D // 2

NC, NS = 2, 16
EP = 327680
NP = 10240
BI = 128
RW = EP // BI // NS
SCH = 128
RPS = SCH // BI
NCH = RW // RPS
RPT = NP // NS


BE = 2560


def _pe_body(ew_ref, w1_ref, w2_ref, out_ref):
    ew = ew_ref[...]
    w1 = w1_ref[...]
    h = ew[:, 0:1] * w1[0:1, :] + ew[:, 1:2] * w1[1:2, :]
    h = jnp.maximum(h, 0.0)
    out_ref[0] = (
        jnp.dot(h, w2_ref[0], preferred_element_type=jnp.float32) + 1.0
    )


def _pe_call(edge_w, W1, W2):
    return pl.pallas_call(
        _pe_body,
        grid=(EP // BE, NC),
        in_specs=[
            pl.BlockSpec((BE, 2), lambda i, h: (i, 0)),
            pl.BlockSpec((2, DH), lambda i, h: (0, 0)),
            pl.BlockSpec((1, DH, DH), lambda i, h: (h, 0, 0)),
        ],
        out_specs=pl.BlockSpec((1, BE, DH), lambda i, h: (h, i, 0)),
        out_shape=jax.ShapeDtypeStruct((NC, EP, DH), jnp.float32),
    )(edge_w, W1, W2)


def _sc_body(x_hbm, src_hbm, dst_hbm, pe_hbm, acc_hbm, cnt_hbm,
             src_v, dst_v, pe_a, pe_b, xr_a, xr_b, cnt_v, acc_sh,
             sem_a, sem_b):
    c = lax.axis_index("c")
    s = lax.axis_index("s")

    zeros16 = jnp.zeros((16,), jnp.float32)
    ones16 = jnp.ones((16,), jnp.float32)

    @pl.loop(0, NP // 16)
    def _(i):
        cnt_v[pl.ds(i * 16, 16)] = zeros16

    @pl.loop(0, SCH)
    def _(r):
        for g in range(DH // 16):
            xr_a[r, pl.ds(g * 16, 16)] = zeros16

    for q in range(RPT // SCH):
        pltpu.sync_copy(xr_a, acc_sh.at[pl.ds(s * RPT + q * SCH, SCH)])

    pltpu.sync_copy(src_hbm.at[pl.ds(s * RW, RW)], src_v)
    pltpu.sync_copy(dst_hbm.at[pl.ds(s * RW, RW)], dst_v)

    @pl.loop(0, RW)
    def _(r):
        for g in range(BI // 16):
            sl = pl.ds(g * 16, 16)
            src_v[r, sl] = src_v[r, sl] + c * N

    plsc.subcore_barrier()

    def fire(k, pe_buf, xr_buf, sem):
        for t in range(RPS):
            pltpu.async_copy(x_hbm.at[src_v.at[k * RPS + t]],
                             xr_buf.at[pl.ds(t * BI, BI)], sem)
        pltpu.async_copy(pe_hbm.at[c, pl.ds((s * RW + k * RPS) * BI, SCH)],
                         pe_buf, sem)

    def drain(pe_buf, xr_buf, sem):
        for t in range(RPS):
            pltpu.make_async_copy(x_hbm.at[src_v.at[0]],
                                  xr_buf.at[pl.ds(t * BI, BI)], sem).wait()
        pltpu.make_async_copy(pe_hbm.at[c, pl.ds(0, SCH)], pe_buf,
                              sem).wait()

    def compute(k, pe_buf, xr_buf):
        @plsc.parallel_loop(0, SCH, 1, unroll=8)
        def _(r):
            for g in range(DH // 16):
                sl = pl.ds(g * 16, 16)
                xr_buf[r, sl] = xr_buf[r, sl] * pe_buf[r, sl]

        for t in range(RPS):
            pltpu.sync_copy(xr_buf.at[pl.ds(t * BI, BI)],
                            acc_sh.at[dst_v.at[k * RPS + t]], add=True)

        @pl.when(c == 0)
        def _():
            for t in range(RPS):
                for g in range(BI // 16):
                    idx16 = dst_v[k * RPS + t, pl.ds(g * 16, 16)]
                    plsc.addupdate_scatter(cnt_v, [idx16], ones16)

    fire(0, pe_a, xr_a, sem_a)

    @pl.loop(0, NCH // 2)
    def _(kk):
        ka = 2 * kk
        fire(ka + 1, pe_b, xr_b, sem_b)
        drain(pe_a, xr_a, sem_a)
        compute(ka, pe_a, xr_a)

        @pl.when(kk < NCH // 2 - 1)
        def _():
            fire(ka + 2, pe_a, xr_a, sem_a)

        drain(pe_b, xr_b, sem_b)
        compute(ka + 1, pe_b, xr_b)

    plsc.subcore_barrier()

    pltpu.sync_copy(acc_sh.at[pl.ds(s * RPT, RPT)],
                    acc_hbm.at[c, pl.ds(s * RPT, RPT)])

    @pl.when(c == 0)
    def _():
        pltpu.sync_copy(cnt_v, cnt_hbm.at[pl.ds(s * NP, NP)])


_sc_call = pl.kernel(
    _sc_body,
    out_type=[
        jax.ShapeDtypeStruct((NC, NP, DH), jnp.float32),
        jax.ShapeDtypeStruct((NS * NP,), jnp.float32),
    ],
    mesh=plsc.VectorSubcoreMesh(core_axis_name="c", subcore_axis_name="s"),
    compiler_params=pltpu.CompilerParams(needs_layout_passes=False,
                                         use_tc_tiling_on_sc=False),
    scratch_types=[
        pltpu.VMEM((RW, BI), jnp.int32),
        pltpu.VMEM((RW, BI), jnp.int32),
        pltpu.VMEM((SCH, DH), jnp.float32),
        pltpu.VMEM((SCH, DH), jnp.float32),
        pltpu.VMEM((SCH, DH), jnp.float32),
        pltpu.VMEM((SCH, DH), jnp.float32),
        pltpu.VMEM((NP,), jnp.float32),
        pltpu.VMEM_SHARED((NP, DH), jnp.float32),
        pltpu.SemaphoreType.DMA,
        pltpu.SemaphoreType.DMA,
    ],
)


BN = 2000
NB = N // BN


def _fin_body(acc_ref, cnt_ref, x_ref, w_ref, b_ref, g_ref, be_ref,
              out_ref, t_sc, s1_sc, s2_sc):
    p = pl.program_id(0)
    i = pl.program_id(1)

    @pl.when(p == 0)
    def _():
        xb = x_ref[...]
        ssum = jnp.concatenate([acc_ref[0], acc_ref[1]], axis=1) + xb
        cnt = jnp.sum(cnt_ref[...], axis=1) + 1.0
        mean = ssum / cnt[:, None]
        wt = w_ref[...]
        t_pre = (
            jnp.dot(xb, wt[:D], preferred_element_type=jnp.float32)
            + jnp.dot(mean, wt[D:], preferred_element_type=jnp.float32)
            + b_ref[...][None, :]
        )
        nrm = jnp.sqrt(jnp.sum(t_pre * t_pre, axis=1, keepdims=True))
        t = t_pre / jnp.maximum(nrm, 1e-12)
        t_sc[pl.ds(i * BN, BN), :] = t

        @pl.when(i == 0)
        def _():
            s1_sc[...] = jnp.zeros_like(s1_sc)
            s2_sc[...] = jnp.zeros_like(s2_sc)

        s1_sc[...] += jnp.sum(t, axis=0, keepdims=True)
        s2_sc[...] += jnp.sum(t * t, axis=0, keepdims=True)

    @pl.when(p == 1)
    def _():
        t = t_sc[pl.ds(i * BN, BN), :]
        mu = s1_sc[...] / N
        var = s2_sc[...] / N - mu * mu
        y = (t - mu) * lax.rsqrt(var + 1e-5) * g_ref[...][None, :] \
            + be_ref[...][None, :]
        out_ref[...] = jnp.maximum(y + x_ref[...], 0.0)


def _fin_call(acc, cntp, x, W, b, gamma, beta):
    return pl.pallas_call(
        _fin_body,
        grid=(2, NB),
        in_specs=[
            pl.BlockSpec((NC, BN, DH), lambda p, i: (0, i, 0)),
            pl.BlockSpec((BN, NS), lambda p, i: (i, 0)),
            pl.BlockSpec((BN, D), lambda p, i: (i, 0)),
            pl.BlockSpec((2 * D, D), lambda p, i: (0, 0)),
            pl.BlockSpec((D,), lambda p, i: (0,)),
            pl.BlockSpec((D,), lambda p, i: (0,)),
            pl.BlockSpec((D,), lambda p, i: (0,)),
        ],
        out_specs=pl.BlockSpec((BN, D), lambda p, i: (i, 0)),
        out_shape=jax.ShapeDtypeStruct((N, D), jnp.float32),
        scratch_shapes=[
            pltpu.VMEM((N, D), jnp.float32),
            pltpu.VMEM((1, D), jnp.float32),
            pltpu.VMEM((1, D), jnp.float32),
        ],
    )(acc, cntp, x, W, b, gamma, beta)


def kernel(x, edge_index, edge_w, W1, W2, W, b, gamma, beta):
    pad = EP - E
    src2d = jnp.concatenate(
        [edge_index[0], jnp.zeros((pad,), jnp.int32)]).reshape(EP // BI, BI)
    dst2d = jnp.concatenate(
        [edge_index[1], jnp.full((pad,), N, jnp.int32)]).reshape(EP // BI, BI)
    ew_pad = jnp.concatenate([edge_w, jnp.zeros((pad, 2), jnp.float32)])
    xcat = jnp.concatenate([x[:, :DH], x[:, DH:]], axis=0)
    w2s = jnp.stack([W2[:, :DH], W2[:, DH:]])
    pe1 = _pe_call(ew_pad, W1, w2s)
    acc, cntp = _sc_call(xcat, src2d, dst2d, pe1)
    return _fin_call(acc, cntp.reshape(NS, NP).T, x, W, b, gamma, beta)


# speedup vs baseline: 3.2572x; 1.3347x over previous
import jax
import jax.numpy as jnp
from jax import lax
from jax.experimental import pallas as pl
from jax.experimental.pallas import tpu as pltpu
from jax.experimental.pallas import tpu_sc as plsc

N = 10000
E = 320000
D = 128
DH = D // 2

NC, NS = 2, 16
EP = 327680
NP = 10240
BI = 128
RW = EP // BI // NS
SCH = 128
RPS = SCH // BI
NCH = RW // RPS
RPT = NP // NS


EPH = EP // 2
BEH = 2048


def _pe_body(ew_ref, w1_ref, w2_ref, out_ref):
    hp = lax.dot_general(ew_ref[...], w1_ref[...],
                         (((0,), (0,)), ((), ())),
                         preferred_element_type=jnp.float32)
    hp = jnp.maximum(hp, 0.0)
    out_ref[0] = (
        jnp.dot(hp, w2_ref[0], preferred_element_type=jnp.float32) + 1.0
    )


def _pe_call(ew4t, w1p, w2d):
    return pl.pallas_call(
        _pe_body,
        grid=(EPH // BEH, NC),
        in_specs=[
            pl.BlockSpec((4, BEH), lambda i, h: (0, i)),
            pl.BlockSpec((4, D), lambda i, h: (0, 0)),
            pl.BlockSpec((1, D, D), lambda i, h: (h, 0, 0)),
        ],
        out_specs=pl.BlockSpec((1, BEH, D), lambda i, h: (h, i, 0)),
        out_shape=jax.ShapeDtypeStruct((NC, EPH, D), jnp.float32),
    )(ew4t, w1p, w2d)


def _sc_body(x_hbm, src_hbm, dst_hbm, pe_hbm, acc_hbm, cnt_hbm,
             src_v, dst_v, pe_a, pe_b, xr_a, xr_b, cnt_v, acc_sh,
             sem_a, sem_b):
    c = lax.axis_index("c")
    s = lax.axis_index("s")

    zeros16 = jnp.zeros((16,), jnp.float32)
    ones16 = jnp.ones((16,), jnp.float32)

    @pl.loop(0, NP // 16)
    def _(i):
        cnt_v[pl.ds(i * 16, 16)] = zeros16

    @pl.loop(0, SCH)
    def _(r):
        for g in range(DH // 16):
            xr_a[r, pl.ds(g * 16, 16)] = zeros16

    for q in range(RPT // SCH):
        pltpu.sync_copy(xr_a, acc_sh.at[pl.ds(s * RPT + q * SCH, SCH)])

    pltpu.sync_copy(src_hbm.at[pl.ds(s * RW, RW)], src_v)
    pltpu.sync_copy(dst_hbm.at[pl.ds(s * RW, RW)], dst_v)

    @pl.loop(0, RW)
    def _(r):
        for g in range(BI // 16):
            sl = pl.ds(g * 16, 16)
            src_v[r, sl] = src_v[r, sl] + c * N

    plsc.subcore_barrier()

    def fire(k, pe_buf, xr_buf, sem):
        for t in range(RPS):
            pltpu.async_copy(x_hbm.at[src_v.at[k * RPS + t]],
                             xr_buf.at[pl.ds(t * BI, BI)], sem)
        pltpu.async_copy(pe_hbm.at[c, pl.ds((s * RW + k * RPS) * (BI // 2),
                                            SCH // 2)],
                         pe_buf, sem)

    def drain(pe_buf, xr_buf, sem):
        for t in range(RPS):
            pltpu.make_async_copy(x_hbm.at[src_v.at[0]],
                                  xr_buf.at[pl.ds(t * BI, BI)], sem).wait()
        pltpu.make_async_copy(pe_hbm.at[c, pl.ds(0, SCH // 2)], pe_buf,
                              sem).wait()

    def compute(k, pe_buf, xr_buf):
        @plsc.parallel_loop(0, SCH // 2, 1, unroll=4)
        def _(r):
            for par in range(2):
                for g in range(DH // 16):
                    xsl = pl.ds(g * 16, 16)
                    psl = pl.ds(par * DH + g * 16, 16)
                    xr_buf[2 * r + par, xsl] = \
                        xr_buf[2 * r + par, xsl] * pe_buf[r, psl]

        for t in range(RPS):
            pltpu.sync_copy(xr_buf.at[pl.ds(t * BI, BI)],
                            acc_sh.at[dst_v.at[k * RPS + t]], add=True)

        @pl.when(c == 0)
        def _():
            for t in range(RPS):
                for g in range(BI // 16):
                    idx16 = dst_v[k * RPS + t, pl.ds(g * 16, 16)]
                    plsc.addupdate_scatter(cnt_v, [idx16], ones16)

    fire(0, pe_a, xr_a, sem_a)

    @pl.loop(0, NCH // 2)
    def _(kk):
        ka = 2 * kk
        fire(ka + 1, pe_b, xr_b, sem_b)
        drain(pe_a, xr_a, sem_a)
        compute(ka, pe_a, xr_a)

        @pl.when(kk < NCH // 2 - 1)
        def _():
            fire(ka + 2, pe_a, xr_a, sem_a)

        drain(pe_b, xr_b, sem_b)
        compute(ka + 1, pe_b, xr_b)

    plsc.subcore_barrier()

    pltpu.sync_copy(acc_sh.at[pl.ds(s * RPT, RPT)],
                    acc_hbm.at[c, pl.ds(s * RPT, RPT)])

    @pl.when(c == 0)
    def _():
        pltpu.sync_copy(cnt_v, cnt_hbm.at[pl.ds(s * NP, NP)])


_sc_call = pl.kernel(
    _sc_body,
    out_type=[
        jax.ShapeDtypeStruct((NC, NP, DH), jnp.float32),
        jax.ShapeDtypeStruct((NS * NP,), jnp.float32),
    ],
    mesh=plsc.VectorSubcoreMesh(core_axis_name="c", subcore_axis_name="s"),
    compiler_params=pltpu.CompilerParams(needs_layout_passes=False,
                                         use_tc_tiling_on_sc=False),
    scratch_types=[
        pltpu.VMEM((RW, BI), jnp.int32),
        pltpu.VMEM((RW, BI), jnp.int32),
        pltpu.VMEM((SCH // 2, D), jnp.float32),
        pltpu.VMEM((SCH // 2, D), jnp.float32),
        pltpu.VMEM((SCH, DH), jnp.float32),
        pltpu.VMEM((SCH, DH), jnp.float32),
        pltpu.VMEM((NP,), jnp.float32),
        pltpu.VMEM_SHARED((NP, DH), jnp.float32),
        pltpu.SemaphoreType.DMA,
        pltpu.SemaphoreType.DMA,
    ],
)


BN = 2000
NB = N // BN


def _fin_body(acc_ref, cnt_ref, x_ref, w_ref, b_ref, g_ref, be_ref,
              out_ref, t_sc, s1_sc, s2_sc):
    p = pl.program_id(0)
    i = pl.program_id(1)

    @pl.when(p == 0)
    def _():
        xb = x_ref[...]
        ssum = jnp.concatenate([acc_ref[0], acc_ref[1]], axis=1) + xb
        cnt = jnp.sum(cnt_ref[...], axis=1) + 1.0
        mean = ssum / cnt[:, None]
        wt = w_ref[...]
        t_pre = (
            jnp.dot(xb, wt[:D], preferred_element_type=jnp.float32)
            + jnp.dot(mean, wt[D:], preferred_element_type=jnp.float32)
            + b_ref[...][None, :]
        )
        nrm = jnp.sqrt(jnp.sum(t_pre * t_pre, axis=1, keepdims=True))
        t = t_pre / jnp.maximum(nrm, 1e-12)
        t_sc[pl.ds(i * BN, BN), :] = t

        @pl.when(i == 0)
        def _():
            s1_sc[...] = jnp.zeros_like(s1_sc)
            s2_sc[...] = jnp.zeros_like(s2_sc)

        s1_sc[...] += jnp.sum(t, axis=0, keepdims=True)
        s2_sc[...] += jnp.sum(t * t, axis=0, keepdims=True)

    @pl.when(p == 1)
    def _():
        t = t_sc[pl.ds(i * BN, BN), :]
        mu = s1_sc[...] / N
        var = s2_sc[...] / N - mu * mu
        y = (t - mu) * lax.rsqrt(var + 1e-5) * g_ref[...][None, :] \
            + be_ref[...][None, :]
        out_ref[...] = jnp.maximum(y + x_ref[...], 0.0)


def _fin_call(acc, cntp, x, W, b, gamma, beta):
    return pl.pallas_call(
        _fin_body,
        grid=(2, NB),
        in_specs=[
            pl.BlockSpec((NC, BN, DH), lambda p, i: (0, i, 0)),
            pl.BlockSpec((BN, NS), lambda p, i: (i, 0)),
            pl.BlockSpec((BN, D), lambda p, i: (i, 0)),
            pl.BlockSpec((2 * D, D), lambda p, i: (0, 0)),
            pl.BlockSpec((D,), lambda p, i: (0,)),
            pl.BlockSpec((D,), lambda p, i: (0,)),
            pl.BlockSpec((D,), lambda p, i: (0,)),
        ],
        out_specs=pl.BlockSpec((BN, D), lambda p, i: (i, 0)),
        out_shape=jax.ShapeDtypeStruct((N, D), jnp.float32),
        scratch_shapes=[
            pltpu.VMEM((N, D), jnp.float32),
            pltpu.VMEM((1, D), jnp.float32),
            pltpu.VMEM((1, D), jnp.float32),
        ],
    )(acc, cntp, x, W, b, gamma, beta)


def kernel(x, edge_index, edge_w, W1, W2, W, b, gamma, beta):
    pad = EP - E
    src2d = jnp.concatenate(
        [edge_index[0], jnp.zeros((pad,), jnp.int32)]).reshape(EP // BI, BI)
    dst2d = jnp.concatenate(
        [edge_index[1], jnp.full((pad,), N, jnp.int32)]).reshape(EP // BI, BI)
    ew4t = jnp.concatenate(
        [edge_w, jnp.zeros((pad, 2), jnp.float32)]).reshape(EPH, 4).T
    xcat = jnp.concatenate([x[:, :DH], x[:, DH:]], axis=0)
    z2 = jnp.zeros((2, DH), jnp.float32)
    w1p = jnp.concatenate(
        [jnp.concatenate([W1, z2], axis=1),
         jnp.concatenate([z2, W1], axis=1)], axis=0)
    zd = jnp.zeros((DH, DH), jnp.float32)
    w2d = jnp.stack([
        jnp.concatenate(
            [jnp.concatenate([W2[:, c * DH:(c + 1) * DH], zd], axis=1),
             jnp.concatenate([zd, W2[:, c * DH:(c + 1) * DH]], axis=1)],
            axis=0)
        for c in range(NC)])
    pe1 = _pe_call(ew4t, w1p, w2d)
    acc, cntp = _sc_call(xcat, src2d, dst2d, pe1)
    return _fin_call(acc, cntp.reshape(NS, NP).T, x, W, b, gamma, beta)


# speedup vs baseline: 5.0794x; 1.5595x over previous
import jax
import jax.numpy as jnp
from jax import lax
from jax.experimental import pallas as pl
from jax.experimental.pallas import tpu as pltpu
from jax.experimental.pallas import tpu_sc as plsc

N = 10000
E = 320000
D = 128
DH = D // 2

NC, NS = 2, 16
EP = 327680
NP = 10240
BI = 128
RW = EP // BI // NS
SCH = 128
RPS = SCH // BI
NCH = RW // RPS
RPT = NP // NS


EPH = EP // 2
BEH = 2048


def _pe_body(ew_ref, w1_ref, w2_ref, out_ref):
    hp = lax.dot_general(ew_ref[...], w1_ref[...],
                         (((0,), (0,)), ((), ())),
                         preferred_element_type=jnp.float32)
    hp = jnp.maximum(hp, 0.0)
    out_ref[0] = (
        jnp.dot(hp, w2_ref[0], preferred_element_type=jnp.float32) + 1.0
    )


def _pe_call(ew4t, w1p, w2d):
    return pl.pallas_call(
        _pe_body,
        grid=(EPH // BEH, NC),
        in_specs=[
            pl.BlockSpec((4, BEH), lambda i, h: (0, i)),
            pl.BlockSpec((4, D), lambda i, h: (0, 0)),
            pl.BlockSpec((1, D, D), lambda i, h: (h, 0, 0)),
        ],
        out_specs=pl.BlockSpec((1, BEH, D), lambda i, h: (h, i, 0)),
        out_shape=jax.ShapeDtypeStruct((NC, EPH, D), jnp.float32),
    )(ew4t, w1p, w2d)


def _sc_body(x_hbm, src_hbm, dst_hbm, pe_hbm, acc_hbm, cnt_hbm,
             src_v, dst_v, pe_a, pe_b, xr_a, xr_b, cnt_v, acc_sh,
             sem_a, sem_b):
    c = lax.axis_index("c")
    s = lax.axis_index("s")

    zeros16 = jnp.zeros((16,), jnp.float32)
    ones16 = jnp.ones((16,), jnp.float32)

    @pl.loop(0, NP // 16)
    def _(i):
        cnt_v[pl.ds(i * 16, 16)] = zeros16

    @pl.loop(0, SCH)
    def _(r):
        for g in range(DH // 16):
            xr_a[r, pl.ds(g * 16, 16)] = zeros16

    for q in range(RPT // SCH):
        pltpu.sync_copy(xr_a, acc_sh.at[pl.ds(s * RPT + q * SCH, SCH)])

    pltpu.sync_copy(src_hbm.at[pl.ds(s * RW, RW)], src_v)
    pltpu.sync_copy(dst_hbm.at[pl.ds(s * RW, RW)], dst_v)

    @pl.loop(0, RW)
    def _(r):
        for g in range(BI // 16):
            sl = pl.ds(g * 16, 16)
            src_v[r, sl] = src_v[r, sl] + c * N

    plsc.subcore_barrier()

    shalf = s // 8
    coff = shalf * DH

    def fire(k, pe_buf, xr_buf, sem):
        for t in range(RPS):
            pltpu.async_copy(x_hbm.at[src_v.at[k * RPS + t]],
                             xr_buf.at[pl.ds(t * BI, BI)], sem)
        prow = (s * RW + k * RPS) * BI - shalf * EPH
        pltpu.async_copy(pe_hbm.at[c, pl.ds(prow, SCH), pl.ds(coff, DH)],
                         pe_buf, sem)

    def drain(pe_buf, xr_buf, sem):
        for t in range(RPS):
            pltpu.make_async_copy(x_hbm.at[src_v.at[0]],
                                  xr_buf.at[pl.ds(t * BI, BI)], sem).wait()
        pltpu.make_async_copy(pe_hbm.at[c, pl.ds(0, SCH), pl.ds(0, DH)],
                              pe_buf, sem).wait()

    def compute(k, pe_buf, xr_buf):
        @plsc.parallel_loop(0, SCH, 1, unroll=4)
        def _(r):
            for g in range(DH // 16):
                sl = pl.ds(g * 16, 16)
                xr_buf[r, sl] = xr_buf[r, sl] * pe_buf[r, sl]

        for t in range(RPS):
            pltpu.sync_copy(xr_buf.at[pl.ds(t * BI, BI)],
                            acc_sh.at[dst_v.at[k * RPS + t]], add=True)

        @pl.when(c == 0)
        def _():
            for t in range(RPS):
                for g in range(BI // 16):
                    idx16 = dst_v[k * RPS + t, pl.ds(g * 16, 16)]
                    plsc.addupdate_scatter(cnt_v, [idx16], ones16)

    fire(0, pe_a, xr_a, sem_a)

    @pl.loop(0, NCH // 2)
    def _(kk):
        ka = 2 * kk
        fire(ka + 1, pe_b, xr_b, sem_b)
        drain(pe_a, xr_a, sem_a)
        compute(ka, pe_a, xr_a)

        @pl.when(kk < NCH // 2 - 1)
        def _():
            fire(ka + 2, pe_a, xr_a, sem_a)

        drain(pe_b, xr_b, sem_b)
        compute(ka + 1, pe_b, xr_b)

    plsc.subcore_barrier()

    pltpu.sync_copy(acc_sh.at[pl.ds(s * RPT, RPT)],
                    acc_hbm.at[c, pl.ds(s * RPT, RPT)])

    @pl.when(c == 0)
    def _():
        pltpu.sync_copy(cnt_v, cnt_hbm.at[pl.ds(s * NP, NP)])


_sc_call = pl.kernel(
    _sc_body,
    out_type=[
        jax.ShapeDtypeStruct((NC, NP, DH), jnp.float32),
        jax.ShapeDtypeStruct((NS * NP,), jnp.float32),
    ],
    mesh=plsc.VectorSubcoreMesh(core_axis_name="c", subcore_axis_name="s"),
    compiler_params=pltpu.CompilerParams(needs_layout_passes=False,
                                         use_tc_tiling_on_sc=False),
    scratch_types=[
        pltpu.VMEM((RW, BI), jnp.int32),
        pltpu.VMEM((RW, BI), jnp.int32),
        pltpu.VMEM((SCH, DH), jnp.float32),
        pltpu.VMEM((SCH, DH), jnp.float32),
        pltpu.VMEM((SCH, DH), jnp.float32),
        pltpu.VMEM((SCH, DH), jnp.float32),
        pltpu.VMEM((NP,), jnp.float32),
        pltpu.VMEM_SHARED((NP, DH), jnp.float32),
        pltpu.SemaphoreType.DMA,
        pltpu.SemaphoreType.DMA,
    ],
)


BN = 2000
NB = N // BN


def _fin_body(acc_ref, cnt_ref, x_ref, w_ref, b_ref, g_ref, be_ref,
              out_ref, t_sc, s1_sc, s2_sc):
    p = pl.program_id(0)
    i = pl.program_id(1)

    @pl.when(p == 0)
    def _():
        xb = x_ref[...]
        ssum = jnp.concatenate([acc_ref[0], acc_ref[1]], axis=1) + xb
        cnt = jnp.sum(cnt_ref[...], axis=1) + 1.0
        mean = ssum / cnt[:, None]
        wt = w_ref[...]
        t_pre = (
            jnp.dot(xb, wt[:D], preferred_element_type=jnp.float32)
            + jnp.dot(mean, wt[D:], preferred_element_type=jnp.float32)
            + b_ref[...][None, :]
        )
        nrm = jnp.sqrt(jnp.sum(t_pre * t_pre, axis=1, keepdims=True))
        t = t_pre / jnp.maximum(nrm, 1e-12)
        t_sc[pl.ds(i * BN, BN), :] = t

        @pl.when(i == 0)
        def _():
            s1_sc[...] = jnp.zeros_like(s1_sc)
            s2_sc[...] = jnp.zeros_like(s2_sc)

        s1_sc[...] += jnp.sum(t, axis=0, keepdims=True)
        s2_sc[...] += jnp.sum(t * t, axis=0, keepdims=True)

    @pl.when(p == 1)
    def _():
        t = t_sc[pl.ds(i * BN, BN), :]
        mu = s1_sc[...] / N
        var = s2_sc[...] / N - mu * mu
        y = (t - mu) * lax.rsqrt(var + 1e-5) * g_ref[...][None, :] \
            + be_ref[...][None, :]
        out_ref[...] = jnp.maximum(y + x_ref[...], 0.0)


def _fin_call(acc, cntp, x, W, b, gamma, beta):
    return pl.pallas_call(
        _fin_body,
        grid=(2, NB),
        in_specs=[
            pl.BlockSpec((NC, BN, DH), lambda p, i: (0, i, 0)),
            pl.BlockSpec((BN, NS), lambda p, i: (i, 0)),
            pl.BlockSpec((BN, D), lambda p, i: (i, 0)),
            pl.BlockSpec((2 * D, D), lambda p, i: (0, 0)),
            pl.BlockSpec((D,), lambda p, i: (0,)),
            pl.BlockSpec((D,), lambda p, i: (0,)),
            pl.BlockSpec((D,), lambda p, i: (0,)),
        ],
        out_specs=pl.BlockSpec((BN, D), lambda p, i: (i, 0)),
        out_shape=jax.ShapeDtypeStruct((N, D), jnp.float32),
        scratch_shapes=[
            pltpu.VMEM((N, D), jnp.float32),
            pltpu.VMEM((1, D), jnp.float32),
            pltpu.VMEM((1, D), jnp.float32),
        ],
    )(acc, cntp, x, W, b, gamma, beta)


def kernel(x, edge_index, edge_w, W1, W2, W, b, gamma, beta):
    pad = EP - E
    src2d = jnp.concatenate(
        [edge_index[0], jnp.zeros((pad,), jnp.int32)]).reshape(EP // BI, BI)
    dst2d = jnp.concatenate(
        [edge_index[1], jnp.full((pad,), N, jnp.int32)]).reshape(EP // BI, BI)
    ewt = jnp.concatenate(
        [edge_w.T, jnp.zeros((2, pad), jnp.float32)], axis=1)
    ew4t = jnp.concatenate([ewt[:, :EPH], ewt[:, EPH:]], axis=0)
    xcat = jnp.concatenate([x[:, :DH], x[:, DH:]], axis=0)
    z2 = jnp.zeros((2, DH), jnp.float32)
    w1p = jnp.concatenate(
        [jnp.concatenate([W1, z2], axis=1),
         jnp.concatenate([z2, W1], axis=1)], axis=0)
    zd = jnp.zeros((DH, DH), jnp.float32)
    w2d = jnp.stack([
        jnp.concatenate(
            [jnp.concatenate([W2[:, c * DH:(c + 1) * DH], zd], axis=1),
             jnp.concatenate([zd, W2[:, c * DH:(c + 1) * DH]], axis=1)],
            axis=0)
        for c in range(NC)])
    pe1 = _pe_call(ew4t, w1p, w2d)
    acc, cntp = _sc_call(xcat, src2d, dst2d, pe1)
    return _fin_call(acc, cntp.reshape(NS, NP).T, x, W, b, gamma, beta)
